# baseline (speedup 1.0000x reference)
import functools

import jax
import jax.numpy as jnp
from jax import lax
from jax.experimental import pallas as pl
from jax.experimental.pallas import tpu as pltpu
from jax.experimental.pallas import tpu_sc as plsc

_D = 32
_NW = 32
_SQRT_HALF = 0.7071067811865476


def _sc_segment_sums(idx, table, cells, t):
    cpw = cells // _NW
    cw = 54
    chunks = cpw // cw
    rows_w = cw * t

    mesh = plsc.VectorSubcoreMesh(core_axis_name="c", subcore_axis_name="s")

    @functools.partial(
        pl.kernel,
        out_type=jax.ShapeDtypeStruct((cells, _D), jnp.float32),
        mesh=mesh,
        scratch_types=[
            pltpu.VMEM((rows_w,), jnp.int32),
            pltpu.VMEM((rows_w,), jnp.int32),
            pltpu.VMEM((rows_w, _D), jnp.float32),
            pltpu.VMEM((rows_w, _D), jnp.float32),
            pltpu.VMEM((cw, _D), jnp.float32),
            pltpu.SemaphoreType.DMA,
            pltpu.SemaphoreType.DMA,
        ],
        compiler_params=pltpu.CompilerParams(use_tc_tiling_on_sc=False),
    )
    def sc_kernel(table_hbm, idx_hbm, out_hbm,
                  idx_v0, idx_v1, rows_v0, rows_v1, sums_v, sem0, sem1):
        wid = lax.axis_index("s") * 2 + lax.axis_index("c")
        first = wid * chunks

        def load_idx(k, idx_v):
            pltpu.sync_copy(idx_hbm.at[pl.ds((first + k) * rows_w, rows_w)], idx_v)

        def process(k, idx_v, rows_v, sem, prefetch):
            pltpu.make_async_copy(table_hbm.at[idx_v], rows_v, sem).wait()

            @pl.loop(0, cw)
            def _cell(c):
                r0 = c * t
                for h in (0, 16):
                    acc = rows_v[r0, pl.ds(h, 16)]
                    for tt in range(1, t):
                        acc = acc + rows_v[r0 + tt, pl.ds(h, 16)]
                    sums_v[c, pl.ds(h, 16)] = acc

            pltpu.sync_copy(sums_v, out_hbm.at[pl.ds((first + k) * cw, cw)])

            @pl.when(prefetch)
            def _():
                load_idx(k + 2, idx_v)
                pltpu.async_copy(table_hbm.at[idx_v], rows_v, sem)

        load_idx(0, idx_v0)
        pltpu.async_copy(table_hbm.at[idx_v0], rows_v0, sem0)
        load_idx(1, idx_v1)
        pltpu.async_copy(table_hbm.at[idx_v1], rows_v1, sem1)

        @pl.loop(0, chunks // 2)
        def _pair(p):
            k0 = p * 2
            process(k0, idx_v0, rows_v0, sem0, k0 + 2 < chunks)
            process(k0 + 1, idx_v1, rows_v1, sem1, k0 + 3 < chunks)

    return sc_kernel(table, idx)


def _tc_head(sums, w_row, b, cells, inv_t):
    blk = 27648
    grid = cells // blk

    def body(s_ref, w_ref, b_ref, o_ref):
        xm = s_ref[...] * inv_t
        act = 0.5 * xm * (1.0 + lax.erf(xm * _SQRT_HALF))
        o_ref[...] = jnp.sum(act * w_ref[...], axis=1) + b_ref[0]

    return pl.pallas_call(
        body,
        grid=(grid,),
        in_specs=[
            pl.BlockSpec((blk, _D), lambda i: (i, 0)),
            pl.BlockSpec((1, _D), lambda i: (0, 0)),
            pl.BlockSpec(memory_space=pltpu.SMEM),
        ],
        out_specs=pl.BlockSpec((blk,), lambda i: (i,)),
        out_shape=jax.ShapeDtypeStruct((cells,), jnp.float32),
    )(sums, w_row, b)


def kernel(x, table, W, b):
    bsz, r, c, t = x.shape
    cells = bsz * r * c
    idx = x.reshape(-1).astype(jnp.int32)
    sums = _sc_segment_sums(idx, table, cells, t)
    out = _tc_head(sums, W.reshape(1, _D), b.astype(jnp.float32), cells, 1.0 / t)
    return out.reshape(bsz, r, c)


# speedup vs baseline: 1.0149x; 1.0149x over previous
import functools

import jax
import jax.numpy as jnp
from jax import lax
from jax.experimental import pallas as pl
from jax.experimental.pallas import tpu as pltpu
from jax.experimental.pallas import tpu_sc as plsc

_D = 32
_NW = 32
_SQRT_HALF = 0.7071067811865476


def _sc_segment_sums(idx, table, cells, t):
    cpw = cells // _NW
    cw = 96
    chunks = cpw // cw
    rows_w = cw * t

    mesh = plsc.VectorSubcoreMesh(core_axis_name="c", subcore_axis_name="s")

    @functools.partial(
        pl.kernel,
        out_type=jax.ShapeDtypeStruct((cells, _D), jnp.float32),
        mesh=mesh,
        scratch_types=[
            pltpu.VMEM((rows_w,), jnp.int32),
            pltpu.VMEM((rows_w,), jnp.int32),
            pltpu.VMEM((rows_w, _D), jnp.float32),
            pltpu.VMEM((rows_w, _D), jnp.float32),
            pltpu.VMEM((cw, _D), jnp.float32),
            pltpu.SemaphoreType.DMA,
            pltpu.SemaphoreType.DMA,
        ],
        compiler_params=pltpu.CompilerParams(use_tc_tiling_on_sc=False),
    )
    def sc_kernel(table_hbm, idx_hbm, out_hbm,
                  idx_v0, idx_v1, rows_v0, rows_v1, sums_v, sem0, sem1):
        wid = lax.axis_index("s") * 2 + lax.axis_index("c")
        first = wid * chunks

        def load_idx(k, idx_v):
            pltpu.sync_copy(idx_hbm.at[pl.ds((first + k) * rows_w, rows_w)], idx_v)

        def process(k, idx_v, rows_v, sem, prefetch):
            pltpu.make_async_copy(table_hbm.at[idx_v], rows_v, sem).wait()

            @pl.loop(0, cw)
            def _cell(c):
                r0 = c * t
                for h in (0, 16):
                    acc = rows_v[r0, pl.ds(h, 16)]
                    for tt in range(1, t):
                        acc = acc + rows_v[r0 + tt, pl.ds(h, 16)]
                    sums_v[c, pl.ds(h, 16)] = acc

            pltpu.sync_copy(sums_v, out_hbm.at[pl.ds((first + k) * cw, cw)])

            if prefetch is not False:
                @pl.when(prefetch)
                def _():
                    load_idx(k + 2, idx_v)
                    pltpu.async_copy(table_hbm.at[idx_v], rows_v, sem)

        load_idx(0, idx_v0)
        pltpu.async_copy(table_hbm.at[idx_v0], rows_v0, sem0)
        load_idx(1, idx_v1)
        pltpu.async_copy(table_hbm.at[idx_v1], rows_v1, sem1)

        @pl.loop(0, chunks // 2)
        def _pair(p):
            k0 = p * 2
            process(k0, idx_v0, rows_v0, sem0, k0 + 2 < chunks)
            process(k0 + 1, idx_v1, rows_v1, sem1, k0 + 3 < chunks)

        if chunks % 2:
            process(chunks - 1, idx_v0, rows_v0, sem0, False)

    return sc_kernel(table, idx)


def _tc_head(sums, w_row, b, cells, inv_t):
    blk = 27648
    grid = cells // blk

    def body(s_ref, w_ref, b_ref, o_ref):
        xm = s_ref[...] * inv_t
        act = 0.5 * xm * (1.0 + lax.erf(xm * _SQRT_HALF))
        o_ref[...] = jnp.sum(act * w_ref[...], axis=1) + b_ref[0]

    return pl.pallas_call(
        body,
        grid=(grid,),
        in_specs=[
            pl.BlockSpec((blk, _D), lambda i: (i, 0)),
            pl.BlockSpec((1, _D), lambda i: (0, 0)),
            pl.BlockSpec(memory_space=pltpu.SMEM),
        ],
        out_specs=pl.BlockSpec((blk,), lambda i: (i,)),
        out_shape=jax.ShapeDtypeStruct((cells,), jnp.float32),
    )(sums, w_row, b)


def kernel(x, table, W, b):
    bsz, r, c, t = x.shape
    cells = bsz * r * c
    idx = x.reshape(-1).astype(jnp.int32)
    sums = _sc_segment_sums(idx, table, cells, t)
    out = _tc_head(sums, W.reshape(1, _D), b.astype(jnp.float32), cells, 1.0 / t)
    return out.reshape(bsz, r, c)
